# Initial kernel scaffold; baseline (speedup 1.0000x reference)
import jax
import jax.numpy as jnp
from jax.experimental import pallas as pl


def kernel(couples_pos_emo, doc_sents_he, doc_sents_hc, all_cls, params):
    raise NotImplementedError("write your pallas kernel here")


# speedup vs baseline: 19.0396x; 19.0396x over previous
import functools
import math

import jax
import jax.numpy as jnp
from jax.experimental import pallas as pl


def _fold_kernel(win_ref, w1pp_ref, bin_ref, b1pp_ref, w2pp_ref, wo1_ref,
                 b2pp_ref, bo1_ref, w1allp_ref, b1allp_ref, allcls_ref,
                 wa_ref, ba_ref, wb_ref, bb_ref, c_ref, *, inv_sqrt_p):
    f32 = jnp.float32
    wa_ref[...] = jnp.dot(win_ref[...], w1pp_ref[...], preferred_element_type=f32)
    ba_ref[...] = jnp.dot(bin_ref[...], w1pp_ref[...], preferred_element_type=f32) + b1pp_ref[...]
    wb_ref[...] = jnp.dot(w2pp_ref[...], wo1_ref[...], preferred_element_type=f32)
    bb_ref[...] = jnp.dot(b2pp_ref[...], wo1_ref[...], preferred_element_type=f32) + bo1_ref[...]
    c_ref[...] = (jnp.dot(allcls_ref[...] * inv_sqrt_p, w1allp_ref[...],
                          preferred_element_type=f32) + b1allp_ref[...])


def _chain_kernel(x_ref, wa_ref, ba_ref, c_ref, wb_ref, bb_ref, wo_ref,
                  bo_ref, out_ref):
    f32 = jnp.float32
    x = x_ref[0]
    h = jnp.dot(x, wa_ref[...], preferred_element_type=f32)
    h = jnp.maximum((h + ba_ref[...] + c_ref[0]) * 0.5, 0.0)
    g = jnp.dot(h, wb_ref[...], preferred_element_type=f32)
    g = jnp.maximum(g + bb_ref[...], 0.0)
    out_ref[0] = jnp.dot(g, wo_ref[...], preferred_element_type=f32) + bo_ref[...]


def kernel(couples_pos_emo, doc_sents_he, doc_sents_hc, all_cls, params):
    del doc_sents_he, doc_sents_hc
    f32 = jnp.float32
    B, P, PAIR_IN = couples_pos_emo.shape
    FEAT = all_cls.shape[1]
    OUT = params['Wo1'].shape[0]

    win = params['Win']
    w1pp = params['W1']['pp']
    b_in = params['bin'].reshape(1, -1)
    b1pp = params['b1']['pp'].reshape(1, -1)
    w2pp = params['W2']['pp']
    wo1 = params['Wo1']
    b2pp = params['b2']['pp'].reshape(1, -1)
    bo1 = params['bo1'].reshape(1, -1)
    w1allp = params['W1']['allp']
    b1allp = params['b1']['allp'].reshape(1, -1)

    wa, ba, wb, bb, c = pl.pallas_call(
        functools.partial(_fold_kernel, inv_sqrt_p=1.0 / math.sqrt(P)),
        out_shape=(
            jax.ShapeDtypeStruct((PAIR_IN, FEAT), f32),
            jax.ShapeDtypeStruct((1, FEAT), f32),
            jax.ShapeDtypeStruct((OUT, OUT), f32),
            jax.ShapeDtypeStruct((1, OUT), f32),
            jax.ShapeDtypeStruct((B, FEAT), f32),
        ),
    )(win, w1pp, b_in, b1pp, w2pp, wo1, b2pp, bo1, w1allp, b1allp, all_cls)

    TP = 512
    nt = pl.cdiv(P, TP)
    out = pl.pallas_call(
        _chain_kernel,
        grid=(B, nt),
        in_specs=[
            pl.BlockSpec((1, TP, PAIR_IN), lambda b, t: (b, t, 0)),
            pl.BlockSpec((PAIR_IN, FEAT), lambda b, t: (0, 0)),
            pl.BlockSpec((1, FEAT), lambda b, t: (0, 0)),
            pl.BlockSpec((1, 1, FEAT), lambda b, t: (b, 0, 0)),
            pl.BlockSpec((OUT, OUT), lambda b, t: (0, 0)),
            pl.BlockSpec((1, OUT), lambda b, t: (0, 0)),
            pl.BlockSpec((OUT, 1), lambda b, t: (0, 0)),
            pl.BlockSpec((1, 1), lambda b, t: (0, 0)),
        ],
        out_specs=pl.BlockSpec((1, TP, 1), lambda b, t: (b, t, 0)),
        out_shape=jax.ShapeDtypeStruct((B, P, 1), f32),
    )(couples_pos_emo, wa, ba, c.reshape(B, 1, FEAT), wb, bb, params['Wo2'],
      params['bo2'].reshape(1, 1))

    return out[:, :, 0]


# speedup vs baseline: 24.7215x; 1.2984x over previous
import functools
import math

import jax
import jax.numpy as jnp
from jax.experimental import pallas as pl


def _fold_kernel(win_ref, w1pp_ref, bin_ref, b1pp_ref, w2pp_ref, wo1_ref,
                 b2pp_ref, bo1_ref, w1allp_ref, b1allp_ref, allcls_ref,
                 wa_ref, ba_ref, wb_ref, bb_ref, c_ref, *, inv_sqrt_p):
    f32 = jnp.float32
    wa_ref[...] = jnp.dot(win_ref[...], w1pp_ref[...], preferred_element_type=f32)
    ba_ref[...] = jnp.dot(bin_ref[...], w1pp_ref[...], preferred_element_type=f32) + b1pp_ref[...]
    wb_ref[...] = jnp.dot(w2pp_ref[...], wo1_ref[...], preferred_element_type=f32)
    bb_ref[...] = jnp.dot(b2pp_ref[...], wo1_ref[...], preferred_element_type=f32) + bo1_ref[...]
    c_ref[...] = (jnp.dot(allcls_ref[...] * inv_sqrt_p, w1allp_ref[...],
                          preferred_element_type=f32) + b1allp_ref[...])


def _chain_kernel(x_ref, wa_ref, ba_ref, c_ref, wb_ref, bb_ref, wo_ref,
                  bo_ref, out_ref):
    f32 = jnp.float32
    x = x_ref[0]
    h = jnp.dot(x, wa_ref[...], preferred_element_type=f32)
    h = jnp.maximum((h + ba_ref[...] + c_ref[0]) * 0.5, 0.0)
    g = jnp.dot(h, wb_ref[...], preferred_element_type=f32)
    g = jnp.maximum(g + bb_ref[...], 0.0)
    out_ref[0] = jnp.dot(g, wo_ref[...], preferred_element_type=f32) + bo_ref[...]


def kernel(couples_pos_emo, doc_sents_he, doc_sents_hc, all_cls, params):
    del doc_sents_he, doc_sents_hc
    f32 = jnp.float32
    B, P, PAIR_IN = couples_pos_emo.shape
    FEAT = all_cls.shape[1]
    OUT = params['Wo1'].shape[0]

    win = params['Win']
    w1pp = params['W1']['pp']
    b_in = params['bin'].reshape(1, -1)
    b1pp = params['b1']['pp'].reshape(1, -1)
    w2pp = params['W2']['pp']
    wo1 = params['Wo1']
    b2pp = params['b2']['pp'].reshape(1, -1)
    bo1 = params['bo1'].reshape(1, -1)
    w1allp = params['W1']['allp']
    b1allp = params['b1']['allp'].reshape(1, -1)

    wa, ba, wb, bb, c = pl.pallas_call(
        functools.partial(_fold_kernel, inv_sqrt_p=1.0 / math.sqrt(P)),
        out_shape=(
            jax.ShapeDtypeStruct((PAIR_IN, FEAT), f32),
            jax.ShapeDtypeStruct((1, FEAT), f32),
            jax.ShapeDtypeStruct((OUT, OUT), f32),
            jax.ShapeDtypeStruct((1, OUT), f32),
            jax.ShapeDtypeStruct((B, FEAT), f32),
        ),
    )(win, w1pp, b_in, b1pp, w2pp, wo1, b2pp, bo1, w1allp, b1allp, all_cls)

    TP = 1024
    nt = pl.cdiv(P, TP)
    out = pl.pallas_call(
        _chain_kernel,
        grid=(B, nt),
        in_specs=[
            pl.BlockSpec((1, TP, PAIR_IN), lambda b, t: (b, t, 0)),
            pl.BlockSpec((PAIR_IN, FEAT), lambda b, t: (0, 0)),
            pl.BlockSpec((1, FEAT), lambda b, t: (0, 0)),
            pl.BlockSpec((1, 1, FEAT), lambda b, t: (b, 0, 0)),
            pl.BlockSpec((OUT, OUT), lambda b, t: (0, 0)),
            pl.BlockSpec((1, OUT), lambda b, t: (0, 0)),
            pl.BlockSpec((OUT, 1), lambda b, t: (0, 0)),
            pl.BlockSpec((1, 1), lambda b, t: (0, 0)),
        ],
        out_specs=pl.BlockSpec((1, TP, 1), lambda b, t: (b, t, 0)),
        out_shape=jax.ShapeDtypeStruct((B, P, 1), f32),
    )(couples_pos_emo, wa, ba, c.reshape(B, 1, FEAT), wb, bb, params['Wo2'],
      params['bo2'].reshape(1, 1))

    return out[:, :, 0]


# speedup vs baseline: 29.1309x; 1.1784x over previous
import functools
import math

import jax
import jax.numpy as jnp
from jax.experimental import pallas as pl


def _fold_kernel(win_ref, w1pp_ref, bin_ref, b1pp_ref, w2pp_ref, wo1_ref,
                 b2pp_ref, bo1_ref, w1allp_ref, b1allp_ref, allcls_ref,
                 wa_ref, ba_ref, wb_ref, bb_ref, c_ref, *, inv_sqrt_p):
    f32 = jnp.float32
    wa_ref[...] = jnp.dot(win_ref[...], w1pp_ref[...], preferred_element_type=f32)
    ba_ref[...] = jnp.dot(bin_ref[...], w1pp_ref[...], preferred_element_type=f32) + b1pp_ref[...]
    wb_ref[...] = jnp.dot(w2pp_ref[...], wo1_ref[...], preferred_element_type=f32)
    bb_ref[...] = jnp.dot(b2pp_ref[...], wo1_ref[...], preferred_element_type=f32) + bo1_ref[...]
    c_ref[...] = (jnp.dot(allcls_ref[...] * inv_sqrt_p, w1allp_ref[...],
                          preferred_element_type=f32) + b1allp_ref[...])


def _chain_kernel(x_ref, wa_ref, ba_ref, c_ref, wb_ref, bb_ref, wo_ref,
                  bo_ref, out_ref):
    f32 = jnp.float32
    x = x_ref[0]
    h = jnp.dot(x, wa_ref[...], preferred_element_type=f32)
    h = jnp.maximum((h + ba_ref[...] + c_ref[0]) * 0.5, 0.0)
    g = jnp.dot(h, wb_ref[...], preferred_element_type=f32)
    g = jnp.maximum(g + bb_ref[...], 0.0)
    out_ref[0] = jnp.dot(g, wo_ref[...], preferred_element_type=f32) + bo_ref[...]


def kernel(couples_pos_emo, doc_sents_he, doc_sents_hc, all_cls, params):
    del doc_sents_he, doc_sents_hc
    f32 = jnp.float32
    B, P, PAIR_IN = couples_pos_emo.shape
    FEAT = all_cls.shape[1]
    OUT = params['Wo1'].shape[0]

    win = params['Win']
    w1pp = params['W1']['pp']
    b_in = params['bin'].reshape(1, -1)
    b1pp = params['b1']['pp'].reshape(1, -1)
    w2pp = params['W2']['pp']
    wo1 = params['Wo1']
    b2pp = params['b2']['pp'].reshape(1, -1)
    bo1 = params['bo1'].reshape(1, -1)
    w1allp = params['W1']['allp']
    b1allp = params['b1']['allp'].reshape(1, -1)

    wa, ba, wb, bb, c = pl.pallas_call(
        functools.partial(_fold_kernel, inv_sqrt_p=1.0 / math.sqrt(P)),
        out_shape=(
            jax.ShapeDtypeStruct((PAIR_IN, FEAT), f32),
            jax.ShapeDtypeStruct((1, FEAT), f32),
            jax.ShapeDtypeStruct((OUT, OUT), f32),
            jax.ShapeDtypeStruct((1, OUT), f32),
            jax.ShapeDtypeStruct((B, FEAT), f32),
        ),
    )(win, w1pp, b_in, b1pp, w2pp, wo1, b2pp, bo1, w1allp, b1allp, all_cls)

    TP = 2048
    nt = pl.cdiv(P, TP)
    out = pl.pallas_call(
        _chain_kernel,
        grid=(B, nt),
        in_specs=[
            pl.BlockSpec((1, TP, PAIR_IN), lambda b, t: (b, t, 0)),
            pl.BlockSpec((PAIR_IN, FEAT), lambda b, t: (0, 0)),
            pl.BlockSpec((1, FEAT), lambda b, t: (0, 0)),
            pl.BlockSpec((1, 1, FEAT), lambda b, t: (b, 0, 0)),
            pl.BlockSpec((OUT, OUT), lambda b, t: (0, 0)),
            pl.BlockSpec((1, OUT), lambda b, t: (0, 0)),
            pl.BlockSpec((OUT, 1), lambda b, t: (0, 0)),
            pl.BlockSpec((1, 1), lambda b, t: (0, 0)),
        ],
        out_specs=pl.BlockSpec((1, TP, 1), lambda b, t: (b, t, 0)),
        out_shape=jax.ShapeDtypeStruct((B, P, 1), f32),
    )(couples_pos_emo, wa, ba, c.reshape(B, 1, FEAT), wb, bb, params['Wo2'],
      params['bo2'].reshape(1, 1))

    return out[:, :, 0]


# speedup vs baseline: 31.9605x; 1.0971x over previous
import functools
import math

import jax
import jax.numpy as jnp
from jax.experimental import pallas as pl


def _fold_kernel(win_ref, w1pp_ref, bin_ref, b1pp_ref, w2pp_ref, wo1_ref,
                 b2pp_ref, bo1_ref, w1allp_ref, b1allp_ref, allcls_ref,
                 wa_ref, ba_ref, wb_ref, bb_ref, c_ref, *, inv_sqrt_p):
    f32 = jnp.float32
    wa_ref[...] = jnp.dot(win_ref[...], w1pp_ref[...], preferred_element_type=f32)
    ba_ref[...] = jnp.dot(bin_ref[...], w1pp_ref[...], preferred_element_type=f32) + b1pp_ref[...]
    wb_ref[...] = jnp.dot(w2pp_ref[...], wo1_ref[...], preferred_element_type=f32)
    bb_ref[...] = jnp.dot(b2pp_ref[...], wo1_ref[...], preferred_element_type=f32) + bo1_ref[...]
    c_ref[...] = (jnp.dot(allcls_ref[...] * inv_sqrt_p, w1allp_ref[...],
                          preferred_element_type=f32) + b1allp_ref[...])


def _chain_kernel(x_ref, wa_ref, ba_ref, c_ref, wb_ref, bb_ref, wo_ref,
                  bo_ref, out_ref):
    f32 = jnp.float32
    x = x_ref[0]
    h = jnp.dot(x, wa_ref[...], preferred_element_type=f32)
    h = jnp.maximum((h + ba_ref[...] + c_ref[0]) * 0.5, 0.0)
    g = jnp.dot(h, wb_ref[...], preferred_element_type=f32)
    g = jnp.maximum(g + bb_ref[...], 0.0)
    out_ref[0] = jnp.dot(g, wo_ref[...], preferred_element_type=f32) + bo_ref[...]


def kernel(couples_pos_emo, doc_sents_he, doc_sents_hc, all_cls, params):
    del doc_sents_he, doc_sents_hc
    f32 = jnp.float32
    B, P, PAIR_IN = couples_pos_emo.shape
    FEAT = all_cls.shape[1]
    OUT = params['Wo1'].shape[0]

    win = params['Win']
    w1pp = params['W1']['pp']
    b_in = params['bin'].reshape(1, -1)
    b1pp = params['b1']['pp'].reshape(1, -1)
    w2pp = params['W2']['pp']
    wo1 = params['Wo1']
    b2pp = params['b2']['pp'].reshape(1, -1)
    bo1 = params['bo1'].reshape(1, -1)
    w1allp = params['W1']['allp']
    b1allp = params['b1']['allp'].reshape(1, -1)

    wa, ba, wb, bb, c = pl.pallas_call(
        functools.partial(_fold_kernel, inv_sqrt_p=1.0 / math.sqrt(P)),
        out_shape=(
            jax.ShapeDtypeStruct((PAIR_IN, FEAT), f32),
            jax.ShapeDtypeStruct((1, FEAT), f32),
            jax.ShapeDtypeStruct((OUT, OUT), f32),
            jax.ShapeDtypeStruct((1, OUT), f32),
            jax.ShapeDtypeStruct((B, FEAT), f32),
        ),
    )(win, w1pp, b_in, b1pp, w2pp, wo1, b2pp, bo1, w1allp, b1allp, all_cls)

    TP = 4096
    nt = pl.cdiv(P, TP)
    out = pl.pallas_call(
        _chain_kernel,
        grid=(B, nt),
        in_specs=[
            pl.BlockSpec((1, TP, PAIR_IN), lambda b, t: (b, t, 0)),
            pl.BlockSpec((PAIR_IN, FEAT), lambda b, t: (0, 0)),
            pl.BlockSpec((1, FEAT), lambda b, t: (0, 0)),
            pl.BlockSpec((1, 1, FEAT), lambda b, t: (b, 0, 0)),
            pl.BlockSpec((OUT, OUT), lambda b, t: (0, 0)),
            pl.BlockSpec((1, OUT), lambda b, t: (0, 0)),
            pl.BlockSpec((OUT, 1), lambda b, t: (0, 0)),
            pl.BlockSpec((1, 1), lambda b, t: (0, 0)),
        ],
        out_specs=pl.BlockSpec((1, TP, 1), lambda b, t: (b, t, 0)),
        out_shape=jax.ShapeDtypeStruct((B, P, 1), f32),
    )(couples_pos_emo, wa, ba, c.reshape(B, 1, FEAT), wb, bb, params['Wo2'],
      params['bo2'].reshape(1, 1))

    return out[:, :, 0]


# speedup vs baseline: 32.0951x; 1.0042x over previous
import functools
import math

import jax
import jax.numpy as jnp
from jax.experimental import pallas as pl


def _fold_kernel(win_ref, w1pp_ref, bin_ref, b1pp_ref, w2pp_ref, wo1_ref,
                 b2pp_ref, bo1_ref, w1allp_ref, b1allp_ref, allcls_ref,
                 wa_ref, ba_ref, wb_ref, bb_ref, c_ref, *, inv_sqrt_p):
    f32 = jnp.float32
    wa_ref[...] = jnp.dot(win_ref[...], w1pp_ref[...], preferred_element_type=f32)
    ba_ref[...] = jnp.dot(bin_ref[...], w1pp_ref[...], preferred_element_type=f32) + b1pp_ref[...]
    wb_ref[...] = jnp.dot(w2pp_ref[...], wo1_ref[...], preferred_element_type=f32)
    bb_ref[...] = jnp.dot(b2pp_ref[...], wo1_ref[...], preferred_element_type=f32) + bo1_ref[...]
    c_ref[...] = (jnp.dot(allcls_ref[...] * inv_sqrt_p, w1allp_ref[...],
                          preferred_element_type=f32) + b1allp_ref[...])


def _chain_kernel(x_ref, wa_ref, ba_ref, c_ref, wb_ref, bb_ref, wo_ref,
                  bo_ref, out_ref):
    f32 = jnp.float32
    x = x_ref[0]
    h = jnp.dot(x, wa_ref[...], preferred_element_type=f32)
    h = jnp.maximum((h + ba_ref[...] + c_ref[0]) * 0.5, 0.0)
    g = jnp.dot(h, wb_ref[...], preferred_element_type=f32)
    g = jnp.maximum(g + bb_ref[...], 0.0)
    out_ref[0] = jnp.dot(g, wo_ref[...], preferred_element_type=f32) + bo_ref[...]


def kernel(couples_pos_emo, doc_sents_he, doc_sents_hc, all_cls, params):
    del doc_sents_he, doc_sents_hc
    f32 = jnp.float32
    B, P, PAIR_IN = couples_pos_emo.shape
    FEAT = all_cls.shape[1]
    OUT = params['Wo1'].shape[0]

    win = params['Win']
    w1pp = params['W1']['pp']
    b_in = params['bin'].reshape(1, -1)
    b1pp = params['b1']['pp'].reshape(1, -1)
    w2pp = params['W2']['pp']
    wo1 = params['Wo1']
    b2pp = params['b2']['pp'].reshape(1, -1)
    bo1 = params['bo1'].reshape(1, -1)
    w1allp = params['W1']['allp']
    b1allp = params['b1']['allp'].reshape(1, -1)

    wa, ba, wb, bb, c = pl.pallas_call(
        functools.partial(_fold_kernel, inv_sqrt_p=1.0 / math.sqrt(P)),
        out_shape=(
            jax.ShapeDtypeStruct((PAIR_IN, FEAT), f32),
            jax.ShapeDtypeStruct((1, FEAT), f32),
            jax.ShapeDtypeStruct((OUT, OUT), f32),
            jax.ShapeDtypeStruct((1, OUT), f32),
            jax.ShapeDtypeStruct((B, FEAT), f32),
        ),
    )(win, w1pp, b_in, b1pp, w2pp, wo1, b2pp, bo1, w1allp, b1allp, all_cls)

    TP = 8176
    nt = pl.cdiv(P, TP)
    out = pl.pallas_call(
        _chain_kernel,
        grid=(B, nt),
        in_specs=[
            pl.BlockSpec((1, TP, PAIR_IN), lambda b, t: (b, t, 0)),
            pl.BlockSpec((PAIR_IN, FEAT), lambda b, t: (0, 0)),
            pl.BlockSpec((1, FEAT), lambda b, t: (0, 0)),
            pl.BlockSpec((1, 1, FEAT), lambda b, t: (b, 0, 0)),
            pl.BlockSpec((OUT, OUT), lambda b, t: (0, 0)),
            pl.BlockSpec((1, OUT), lambda b, t: (0, 0)),
            pl.BlockSpec((OUT, 1), lambda b, t: (0, 0)),
            pl.BlockSpec((1, 1), lambda b, t: (0, 0)),
        ],
        out_specs=pl.BlockSpec((1, TP, 1), lambda b, t: (b, t, 0)),
        out_shape=jax.ShapeDtypeStruct((B, P, 1), f32),
    )(couples_pos_emo, wa, ba, c.reshape(B, 1, FEAT), wb, bb, params['Wo2'],
      params['bo2'].reshape(1, 1))

    return out[:, :, 0]


# speedup vs baseline: 32.2990x; 1.0064x over previous
import functools
import math

import jax
import jax.numpy as jnp
from jax.experimental import pallas as pl


def _fold_kernel(win_ref, w1pp_ref, bin_ref, b1pp_ref, w2pp_ref, wo1_ref,
                 b2pp_ref, bo1_ref, w1allp_ref, b1allp_ref, allcls_ref,
                 wa_ref, ba_ref, wb_ref, bb_ref, c_ref, *, inv_sqrt_p):
    f32 = jnp.float32
    wa_ref[...] = jnp.dot(win_ref[...], w1pp_ref[...],
                          preferred_element_type=f32).astype(jnp.bfloat16)
    ba_ref[...] = jnp.dot(bin_ref[...], w1pp_ref[...], preferred_element_type=f32) + b1pp_ref[...]
    wb_ref[...] = jnp.dot(w2pp_ref[...], wo1_ref[...],
                          preferred_element_type=f32).astype(jnp.bfloat16)
    bb_ref[...] = jnp.dot(b2pp_ref[...], wo1_ref[...], preferred_element_type=f32) + bo1_ref[...]
    c_ref[...] = (jnp.dot(allcls_ref[...] * inv_sqrt_p, w1allp_ref[...],
                          preferred_element_type=f32) + b1allp_ref[...])


def _chain_kernel(x_ref, wa_ref, ba_ref, c_ref, wb_ref, bb_ref, wo_ref,
                  bo_ref, out_ref):
    f32 = jnp.float32
    bf16 = jnp.bfloat16
    x = x_ref[0].astype(bf16)
    h = jnp.dot(x, wa_ref[...], preferred_element_type=f32)
    h = jnp.maximum((h + ba_ref[...] + c_ref[0]) * 0.5, 0.0)
    g = jnp.dot(h.astype(bf16), wb_ref[...], preferred_element_type=f32)
    g = jnp.maximum(g + bb_ref[...], 0.0)
    out_ref[0] = jnp.dot(g.astype(bf16), wo_ref[...], preferred_element_type=f32) + bo_ref[...]


def kernel(couples_pos_emo, doc_sents_he, doc_sents_hc, all_cls, params):
    del doc_sents_he, doc_sents_hc
    f32 = jnp.float32
    B, P, PAIR_IN = couples_pos_emo.shape
    FEAT = all_cls.shape[1]
    OUT = params['Wo1'].shape[0]

    win = params['Win']
    w1pp = params['W1']['pp']
    b_in = params['bin'].reshape(1, -1)
    b1pp = params['b1']['pp'].reshape(1, -1)
    w2pp = params['W2']['pp']
    wo1 = params['Wo1']
    b2pp = params['b2']['pp'].reshape(1, -1)
    bo1 = params['bo1'].reshape(1, -1)
    w1allp = params['W1']['allp']
    b1allp = params['b1']['allp'].reshape(1, -1)

    wa, ba, wb, bb, c = pl.pallas_call(
        functools.partial(_fold_kernel, inv_sqrt_p=1.0 / math.sqrt(P)),
        out_shape=(
            jax.ShapeDtypeStruct((PAIR_IN, FEAT), jnp.bfloat16),
            jax.ShapeDtypeStruct((1, FEAT), f32),
            jax.ShapeDtypeStruct((OUT, OUT), jnp.bfloat16),
            jax.ShapeDtypeStruct((1, OUT), f32),
            jax.ShapeDtypeStruct((B, FEAT), f32),
        ),
    )(win, w1pp, b_in, b1pp, w2pp, wo1, b2pp, bo1, w1allp, b1allp, all_cls)

    TP = 8176
    nt = pl.cdiv(P, TP)
    out = pl.pallas_call(
        _chain_kernel,
        grid=(B, nt),
        in_specs=[
            pl.BlockSpec((1, TP, PAIR_IN), lambda b, t: (b, t, 0)),
            pl.BlockSpec((PAIR_IN, FEAT), lambda b, t: (0, 0)),
            pl.BlockSpec((1, FEAT), lambda b, t: (0, 0)),
            pl.BlockSpec((1, 1, FEAT), lambda b, t: (b, 0, 0)),
            pl.BlockSpec((OUT, OUT), lambda b, t: (0, 0)),
            pl.BlockSpec((1, OUT), lambda b, t: (0, 0)),
            pl.BlockSpec((OUT, 1), lambda b, t: (0, 0)),
            pl.BlockSpec((1, 1), lambda b, t: (0, 0)),
        ],
        out_specs=pl.BlockSpec((1, TP, 1), lambda b, t: (b, t, 0)),
        out_shape=jax.ShapeDtypeStruct((B, P, 1), f32),
    )(couples_pos_emo, wa, ba, c.reshape(B, 1, FEAT), wb, bb,
      params['Wo2'].astype(jnp.bfloat16), params['bo2'].reshape(1, 1))

    return out[:, :, 0]
